# baseline (speedup 1.0000x reference)
import functools

import jax
import jax.numpy as jnp
from jax.experimental import pallas as pl
from jax.experimental.pallas import tpu as pltpu

NUM_Q = 4
K = 512
D = 32
ROWS = 64 * 1024
BLOCK = 8192
IDX_PAD = 8


def _rvq_kernel(xt_ref, cbt_ref, cb_ref, qout_ref, idx_ref, loss_ref):
    step = pl.program_id(0)
    nsteps = pl.num_programs(0)

    @pl.when(step == 0)
    def _init():
        loss_ref[...] = jnp.zeros_like(loss_ref)

    x0 = jnp.transpose(xt_ref[...])
    r = x0
    qacc = jnp.zeros_like(x0)
    iota_k = jax.lax.broadcasted_iota(jnp.int32, (K, BLOCK), 0)
    q_iota = jax.lax.broadcasted_iota(jnp.int32, (1, NUM_Q), 1)

    ones3 = jnp.ones((3, BLOCK), dtype=jnp.bfloat16)
    idx_rows = []

    for q in range(NUM_Q):
        cbt = cbt_ref[q]
        cb = cb_ref[q]
        cn = jnp.sum(cb * cb, axis=1, keepdims=True)
        cn_hi = cn.astype(jnp.bfloat16)
        cn_rest = cn - cn_hi.astype(jnp.float32)
        cn_mid = cn_rest.astype(jnp.bfloat16)
        cn_lo = (cn_rest - cn_mid.astype(jnp.float32)).astype(jnp.bfloat16)
        b_aug = jnp.concatenate(
            [cb.astype(jnp.bfloat16) * jnp.bfloat16(-2.0),
             cn_hi, cn_mid, cn_lo], axis=1)
        a_aug = jnp.concatenate(
            [r.astype(jnp.bfloat16), ones3], axis=0)
        dist = jax.lax.dot_general(
            b_aug, a_aug, (((1,), (0,)), ((), ())),
            preferred_element_type=jnp.float32)
        idx = jnp.argmin(dist, axis=0)[None, :]
        onehot = (iota_k == idx).astype(jnp.bfloat16)
        c_hi = cbt.astype(jnp.bfloat16)
        rest = cbt - c_hi.astype(jnp.float32)
        c_mid = rest.astype(jnp.bfloat16)
        c_lo = (rest - c_mid.astype(jnp.float32)).astype(jnp.bfloat16)
        c_cat = jnp.concatenate([c_hi, c_mid, c_lo], axis=0)
        p = jax.lax.dot_general(
            c_cat, onehot, (((1,), (0,)), ((), ())),
            preferred_element_type=jnp.float32)
        q_raw = p[:D] + (p[D:2 * D] + p[2 * D:])
        quant = r + (q_raw - r)
        s = jnp.sum((q_raw - r) * (q_raw - r))
        r = r - quant
        qacc = qacc + quant
        idx_rows.append(idx)
        loss_ref[...] += jnp.where(q_iota == q, s, 0.0)

    qout_ref[...] = jnp.transpose(qacc)
    idx_ref[...] = jnp.transpose(jnp.concatenate(idx_rows, axis=0))

    @pl.when(step == nsteps - 1)
    def _scale():
        loss_ref[...] = loss_ref[...] * (1.25 / float(ROWS * D))


@functools.partial(jax.jit, static_argnames=("interpret",))
def kernel(x, codebooks, interpret=False):
    b, n, d = x.shape
    xf = x.reshape(-1, d)
    cbt = jnp.transpose(codebooks, (0, 2, 1))
    grid = (ROWS // BLOCK,)
    qout, idx, loss = pl.pallas_call(
        _rvq_kernel,
        grid=grid,
        in_specs=[
            pl.BlockSpec((BLOCK, D), lambda i: (i, 0)),
            pl.BlockSpec((NUM_Q, D, K), lambda i: (0, 0, 0)),
            pl.BlockSpec((NUM_Q, K, D), lambda i: (0, 0, 0)),
        ],
        out_specs=[
            pl.BlockSpec((BLOCK, D), lambda i: (i, 0)),
            pl.BlockSpec((BLOCK, NUM_Q), lambda i: (i, 0)),
            pl.BlockSpec((1, NUM_Q), lambda i: (0, 0)),
        ],
        out_shape=[
            jax.ShapeDtypeStruct((ROWS, D), jnp.float32),
            jax.ShapeDtypeStruct((ROWS, NUM_Q), jnp.int32),
            jax.ShapeDtypeStruct((1, NUM_Q), jnp.float32),
        ],
        compiler_params=pltpu.CompilerParams(
            dimension_semantics=("arbitrary",),
        ),
        interpret=interpret,
    )(xf, cbt, codebooks)
    quantized_out = qout.reshape(b, n, d)
    all_indices = idx.reshape(b, n, NUM_Q)
    all_num_expired = jnp.zeros((NUM_Q,), dtype=jnp.int32)
    all_losses = loss.reshape(NUM_Q)
    return (quantized_out, all_indices, all_num_expired, all_losses)


# speedup vs baseline: 1.1093x; 1.1093x over previous
import functools

import jax
import jax.numpy as jnp
from jax.experimental import pallas as pl
from jax.experimental.pallas import tpu as pltpu

NUM_Q = 4
K = 512
D = 32
ROWS = 64 * 1024
BLOCK = 8192
IDX_PAD = 8


def _rvq_kernel(xt_ref, cbt_ref, cb_ref, qout_ref, idx_ref, loss_ref):
    step = pl.program_id(0)
    nsteps = pl.num_programs(0)

    @pl.when(step == 0)
    def _init():
        loss_ref[...] = jnp.zeros_like(loss_ref)

    x0 = xt_ref[...]
    r = x0
    qacc = jnp.zeros_like(x0)
    iota_k = jax.lax.broadcasted_iota(jnp.int32, (K, BLOCK), 0)
    q_iota = jax.lax.broadcasted_iota(jnp.int32, (1, NUM_Q), 1)

    ones3 = jnp.ones((3, BLOCK), dtype=jnp.bfloat16)
    idx_rows = []

    for q in range(NUM_Q):
        cbt = cbt_ref[q]
        cb = cb_ref[q]
        cn = jnp.sum(cb * cb, axis=1, keepdims=True)
        cn_hi = cn.astype(jnp.bfloat16)
        cn_rest = cn - cn_hi.astype(jnp.float32)
        cn_mid = cn_rest.astype(jnp.bfloat16)
        cn_lo = (cn_rest - cn_mid.astype(jnp.float32)).astype(jnp.bfloat16)
        b_aug = jnp.concatenate(
            [cb.astype(jnp.bfloat16) * jnp.bfloat16(-2.0),
             cn_hi, cn_mid, cn_lo], axis=1)
        a_aug = jnp.concatenate(
            [r.astype(jnp.bfloat16), ones3], axis=0)
        dist = jax.lax.dot_general(
            b_aug, a_aug, (((1,), (0,)), ((), ())),
            preferred_element_type=jnp.float32)
        idx = jnp.argmin(dist, axis=0)[None, :]
        onehot = (iota_k == idx).astype(jnp.bfloat16)
        c_hi = cbt.astype(jnp.bfloat16)
        rest = cbt - c_hi.astype(jnp.float32)
        c_mid = rest.astype(jnp.bfloat16)
        c_lo = (rest - c_mid.astype(jnp.float32)).astype(jnp.bfloat16)
        c_cat = jnp.concatenate([c_hi, c_mid, c_lo], axis=0)
        p = jax.lax.dot_general(
            c_cat, onehot, (((1,), (0,)), ((), ())),
            preferred_element_type=jnp.float32)
        q_raw = p[:D] + (p[D:2 * D] + p[2 * D:])
        quant = r + (q_raw - r)
        s = jnp.sum((q_raw - r) * (q_raw - r))
        r = r - quant
        qacc = qacc + quant
        idx_rows.append(idx)
        loss_ref[...] += jnp.where(q_iota == q, s, 0.0)

    qout_ref[...] = qacc
    idx_rows.append(jnp.zeros((IDX_PAD - NUM_Q, BLOCK), dtype=jnp.int32))
    idx_ref[...] = jnp.concatenate(idx_rows, axis=0)

    @pl.when(step == nsteps - 1)
    def _scale():
        loss_ref[...] = loss_ref[...] * (1.25 / float(ROWS * D))


@functools.partial(jax.jit, static_argnames=("interpret",))
def kernel(x, codebooks, interpret=False):
    b, n, d = x.shape
    xt = jnp.transpose(x.reshape(-1, d))
    cbt = jnp.transpose(codebooks, (0, 2, 1))
    grid = (ROWS // BLOCK,)
    qout_t, idx_t, loss = pl.pallas_call(
        _rvq_kernel,
        grid=grid,
        in_specs=[
            pl.BlockSpec((D, BLOCK), lambda i: (0, i)),
            pl.BlockSpec((NUM_Q, D, K), lambda i: (0, 0, 0)),
            pl.BlockSpec((NUM_Q, K, D), lambda i: (0, 0, 0)),
        ],
        out_specs=[
            pl.BlockSpec((D, BLOCK), lambda i: (0, i)),
            pl.BlockSpec((IDX_PAD, BLOCK), lambda i: (0, i)),
            pl.BlockSpec((1, NUM_Q), lambda i: (0, 0)),
        ],
        out_shape=[
            jax.ShapeDtypeStruct((D, ROWS), jnp.float32),
            jax.ShapeDtypeStruct((IDX_PAD, ROWS), jnp.int32),
            jax.ShapeDtypeStruct((1, NUM_Q), jnp.float32),
        ],
        compiler_params=pltpu.CompilerParams(
            dimension_semantics=("arbitrary",),
        ),
        interpret=interpret,
    )(xt, cbt, codebooks)
    quantized_out = jnp.transpose(qout_t).reshape(b, n, d)
    all_indices = jnp.transpose(idx_t[:NUM_Q]).reshape(b, n, NUM_Q)
    all_num_expired = jnp.zeros((NUM_Q,), dtype=jnp.int32)
    all_losses = loss.reshape(NUM_Q)
    return (quantized_out, all_indices, all_num_expired, all_losses)
